# Initial kernel scaffold; baseline (speedup 1.0000x reference)
import jax
import jax.numpy as jnp
from jax.experimental import pallas as pl


def kernel(x, W1, b1, W2, b2, Wu, bu):
    raise NotImplementedError("write your pallas kernel here")


# speedup vs baseline: 1.2418x; 1.2418x over previous
import jax
import jax.numpy as jnp
from jax.experimental import pallas as pl
from jax.experimental.pallas import tpu as pltpu

B, S, D, F, E = 2, 2048, 1024, 4096, 8
T = B * S
BT = 512
NT = T // BT


def _moe_kernel(x_ref, wu_ref, bu_ref, w1_ref, b1_ref, w2_ref, b2_ref,
                out_ref, c_ref):
    t = pl.program_id(0)
    j = pl.program_id(1)

    @pl.when(j == 0)
    def _():
        z = jax.lax.dot_general(
            x_ref[...], wu_ref[...],
            (((1,), (0,)), ((), ())), preferred_element_type=jnp.float32)
        u = jax.nn.sigmoid(z + bu_ref[0, 0])
        n = jnp.clip(jnp.ceil(u * E), 1, E).astype(jnp.int32)
        tok = t * BT + jax.lax.broadcasted_iota(jnp.int32, (BT, E), 0)
        je = jax.lax.broadcasted_iota(jnp.int32, (BT, E), 1)
        k = (je - tok) & (E - 1)
        c_ref[...] = jnp.where(n > k, u / (k + 1).astype(jnp.float32), 0.0)

    oh = (jax.lax.broadcasted_iota(jnp.int32, (1, E), 1) == j).astype(jnp.float32)
    c_col = jnp.sum(c_ref[...] * oh, axis=1, keepdims=True)

    h = jax.lax.dot_general(
        x_ref[...], w1_ref[0],
        (((1,), (0,)), ((), ())), preferred_element_type=jnp.float32)
    h = jnp.maximum(h + b1_ref[0], 0.0)
    hw = (h * c_col).astype(jnp.bfloat16)
    contrib = jax.lax.dot_general(
        hw, w2_ref[0],
        (((1,), (0,)), ((), ())), preferred_element_type=jnp.float32)
    contrib = contrib + c_col * b2_ref[0]

    @pl.when(j == 0)
    def _():
        out_ref[...] = contrib

    @pl.when(j != 0)
    def _():
        out_ref[...] += contrib


@jax.jit
def kernel(x, W1, b1, W2, b2, Wu, bu):
    xb = x.reshape(T, D).astype(jnp.bfloat16)
    w1b = W1.astype(jnp.bfloat16)
    w2b = W2.astype(jnp.bfloat16)
    wub = Wu.astype(jnp.bfloat16)
    bu2 = bu.reshape(1, 1)
    b1r = b1.reshape(E, 1, F)
    b2r = b2.reshape(E, 1, D)
    out = pl.pallas_call(
        _moe_kernel,
        grid=(NT, E),
        in_specs=[
            pl.BlockSpec((BT, D), lambda t, j: (t, 0)),
            pl.BlockSpec((D, 1), lambda t, j: (0, 0)),
            pl.BlockSpec((1, 1), lambda t, j: (0, 0)),
            pl.BlockSpec((1, D, F), lambda t, j: (j, 0, 0)),
            pl.BlockSpec((1, 1, F), lambda t, j: (j, 0, 0)),
            pl.BlockSpec((1, F, D), lambda t, j: (j, 0, 0)),
            pl.BlockSpec((1, 1, D), lambda t, j: (j, 0, 0)),
        ],
        out_specs=pl.BlockSpec((BT, D), lambda t, j: (t, 0)),
        out_shape=jax.ShapeDtypeStruct((T, D), jnp.float32),
        scratch_shapes=[pltpu.VMEM((BT, E), jnp.float32)],
        compiler_params=pltpu.CompilerParams(
            dimension_semantics=("arbitrary", "arbitrary"),
        ),
    )(xb, wub, bu2, w1b, b1r, w2b, b2r)
    return out.reshape(B, S, D)
